# baseline (speedup 1.0000x reference)
import functools

import jax
import jax.numpy as jnp
from jax import lax
from jax.experimental import pallas as pl
from jax.experimental.pallas import tpu as pltpu
from jax.experimental.pallas import tpu_sc as plsc

N_NODES = 10000
N_GRAPHS = 256
NEG_SLOPE = 0.01

_NCORE = 2
_NSUB = 16
_NW = _NCORE * _NSUB
_CHUNK = 128
_ROWS_PT = N_NODES // _NSUB


def _lrelu(v):
    return jnp.where(v >= 0, v, NEG_SLOPE * v)


def _bn(v, g, be):
    mu = jnp.mean(v, axis=0, keepdims=True)
    var = jnp.mean((v - mu) ** 2, axis=0, keepdims=True)
    return g[None, :] * (v - mu) * lax.rsqrt(var + 1e-5) + be[None, :]


def _tc_pre_body(deg_ref, x_ref, w1_ref, dinv_ref, z1p_ref):
    deg = 1.0 + deg_ref[0] + deg_ref[1]
    dinv = lax.rsqrt(deg)
    dinv_ref[...] = dinv
    z1 = jnp.dot(x_ref[...], w1_ref[...], preferred_element_type=jnp.float32)
    z1p_ref[...] = dinv * z1


def _tc_pre(deg, x, W1):
    return pl.pallas_call(
        _tc_pre_body,
        out_shape=(
            jax.ShapeDtypeStruct((N_NODES, 1), jnp.float32),
            jax.ShapeDtypeStruct((N_NODES, W1.shape[1]), jnp.float32),
        ),
    )(deg, x, W1)


def _tc_mid_body(mode_next, s_ref, zp_ref, dinv_ref, W_ref, b_ref, g_ref,
                 be_ref, Wn_ref, out_ref):
    dinv = dinv_ref[...]
    pre = dinv * (s_ref[0] + s_ref[1] + zp_ref[...])
    if W_ref is not None:
        pre = jnp.dot(pre, W_ref[...], preferred_element_type=jnp.float32)
    h = _lrelu(_bn(pre + b_ref[...][None, :], g_ref[...], be_ref[...]))
    if mode_next == "W":
        out_ref[...] = dinv * jnp.dot(h, Wn_ref[...],
                                      preferred_element_type=jnp.float32)
    else:
        out_ref[...] = dinv * h


def _tc_mid(mode_next, s, zp, dinv, W, b, g, be, Wn, dout):
    body = functools.partial(_tc_mid_body, mode_next)
    args = [s, zp, dinv]
    if W is None:
        def body2(s_ref, zp_ref, dinv_ref, b_ref, g_ref, be_ref, *rest):
            if mode_next == "W":
                Wn_ref, out_ref = rest
            else:
                (out_ref,) = rest
                Wn_ref = None
            _tc_mid_body(mode_next, s_ref, zp_ref, dinv_ref, None, b_ref,
                         g_ref, be_ref, Wn_ref, out_ref)
        args += [b, g, be]
        if mode_next == "W":
            args += [Wn]
        return pl.pallas_call(
            body2,
            out_shape=jax.ShapeDtypeStruct((N_NODES, dout), jnp.float32),
        )(*args)
    else:
        def body3(s_ref, zp_ref, dinv_ref, W_ref, b_ref, g_ref, be_ref, *rest):
            if mode_next == "W":
                Wn_ref, out_ref = rest
            else:
                (out_ref,) = rest
                Wn_ref = None
            _tc_mid_body(mode_next, s_ref, zp_ref, dinv_ref, W_ref, b_ref,
                         g_ref, be_ref, Wn_ref, out_ref)
        args += [W, b, g, be]
        if mode_next == "W":
            args += [Wn]
        return pl.pallas_call(
            body3,
            out_shape=jax.ShapeDtypeStruct((N_NODES, dout), jnp.float32),
        )(*args)


def _tc_final_body(s_ref, up_ref, dinv_ref, W5_ref, b5_ref, g5_ref, be5_ref,
                   batch_ref, Wf1_ref, bf1_ref, Wf2_ref, bf2_ref, Wo_ref,
                   bo_ref, out_ref):
    dinv = dinv_ref[...]
    pre = dinv * (s_ref[0] + s_ref[1] + up_ref[...])
    h = jnp.dot(pre, W5_ref[...], preferred_element_type=jnp.float32)
    h = _lrelu(_bn(h + b5_ref[...][None, :], g5_ref[...], be5_ref[...]))
    gids = lax.broadcasted_iota(jnp.int32, (N_GRAPHS, N_NODES), 0)
    mask = (gids == batch_ref[...].reshape(1, N_NODES)).astype(jnp.float32)
    pooled = jnp.dot(mask, h, preferred_element_type=jnp.float32)
    cnt = jnp.sum(mask, axis=1, keepdims=True)
    pooled = pooled / jnp.maximum(cnt, 1.0)
    f = _lrelu(jnp.dot(pooled, Wf1_ref[...],
                       preferred_element_type=jnp.float32) + bf1_ref[...][None, :])
    f = _lrelu(jnp.dot(f, Wf2_ref[...],
                       preferred_element_type=jnp.float32) + bf2_ref[...][None, :])
    out_ref[...] = jnp.dot(f, Wo_ref[...],
                           preferred_element_type=jnp.float32) + bo_ref[...][None, :]


def _tc_final(s, up, dinv, W5, b5, g5, be5, batch, Wf1, bf1, Wf2, bf2, Wo, bo):
    return pl.pallas_call(
        _tc_final_body,
        out_shape=jax.ShapeDtypeStruct((N_GRAPHS, 1), jnp.float32),
    )(s, up, dinv, W5, b5, g5, be5, batch.reshape(1, N_NODES), Wf1, bf1,
      Wf2, bf2, Wo, bo)


_K = 8


@functools.cache
def _sc_edge_agg(d, nc0, nc1):
    _K = 4 if d >= 64 else 8
    assert nc0 % _K == 0 and nc1 % _K == 0
    mesh = plsc.VectorSubcoreMesh(core_axis_name="c", subcore_axis_name="s")

    @functools.partial(
        pl.kernel, mesh=mesh,
        compiler_params=pltpu.CompilerParams(use_tc_tiling_on_sc=False),
        out_type=jax.ShapeDtypeStruct((_NCORE, N_NODES, d), jnp.float32),
        scratch_types=[
            pltpu.VMEM((nc0, _CHUNK), jnp.int32),
            pltpu.VMEM((nc0, _CHUNK), jnp.int32),
            pltpu.VMEM((nc0 * _CHUNK,), jnp.float32),
            [pltpu.VMEM((_CHUNK, d), jnp.float32) for _ in range(_K)],
            pltpu.VMEM_SHARED((N_NODES, d), jnp.float32),
            [pltpu.SemaphoreType.DMA for _ in range(_K)],
            [pltpu.SemaphoreType.DMA for _ in range(_K)],
        ],
    )
    def k(zp_hbm, src_hbm, dst_hbm, w_hbm, zeros_hbm, out_hbm,
          srcs_v, dsts_v, w_v, rowbufs, acc_sh, gsems, ssems):
        c = lax.axis_index("c")
        s = lax.axis_index("s")
        t = c * _NSUB + s
        cp_s = pltpu.async_copy(src_hbm.at[t], srcs_v, gsems[0])
        cp_d = pltpu.async_copy(dst_hbm.at[t], dsts_v, gsems[1])
        cp_w = pltpu.async_copy(w_hbm.at[t], w_v, gsems[2])
        pltpu.sync_copy(zeros_hbm, acc_sh.at[pl.ds(s * 624, 624)])

        @pl.when(s == 0)
        def _():
            pltpu.sync_copy(zeros_hbm.at[pl.ds(0, 16)],
                            acc_sh.at[pl.ds(_NSUB * 624, 16)])
        cp_s.wait()
        cp_d.wait()
        cp_w.wait()
        plsc.subcore_barrier()

        def group_body(gi, carry):
            j0 = gi * _K
            gathers = []
            for b in range(_K):
                gathers.append(pltpu.async_copy(
                    zp_hbm.at[srcs_v.at[j0 + b]], rowbufs[b], gsems[b]))
            scatters = []
            for b in range(_K):
                gathers[b].wait()
                rows_v = rowbufs[b]
                for g in range(_CHUNK // 16):
                    wv16 = w_v[pl.ds((j0 + b) * _CHUNK + g * 16, 16)]
                    for i in range(16):
                        e = g * 16 + i
                        wv = wv16[i]
                        for kk in range(d // 16):
                            rows_v[e, pl.ds(kk * 16, 16)] = (
                                rows_v[e, pl.ds(kk * 16, 16)] * wv)
                scatters.append(pltpu.async_copy(
                    rows_v, acc_sh.at[dsts_v.at[j0 + b]], ssems[b], add=True))
            for b in range(_K):
                scatters[b].wait()
            return carry
        n_groups = jnp.where(c == 0, nc1 // _K, nc0 // _K)
        lax.fori_loop(0, n_groups, group_body, 0)
        plsc.subcore_barrier()
        pltpu.sync_copy(acc_sh.at[pl.ds(s * 624, 624)],
                        out_hbm.at[c, pl.ds(s * 624, 624)])

        @pl.when(s == 0)
        def _():
            pltpu.sync_copy(acc_sh.at[pl.ds(_NSUB * 624, 16)],
                            out_hbm.at[c, pl.ds(_NSUB * 624, 16)])

    return k


@functools.cache
def _sc_deg(nc0, nc1):
    mesh = plsc.VectorSubcoreMesh(core_axis_name="c", subcore_axis_name="s")

    @functools.partial(
        pl.kernel, mesh=mesh,
        compiler_params=pltpu.CompilerParams(use_tc_tiling_on_sc=False),
        out_type=jax.ShapeDtypeStruct((_NCORE, N_NODES, 1), jnp.float32),
        scratch_types=[
            pltpu.VMEM((nc0, _CHUNK), jnp.int32),
            pltpu.VMEM((nc0, _CHUNK, 1), jnp.float32),
            pltpu.VMEM_SHARED((N_NODES, 1), jnp.float32),
            [pltpu.SemaphoreType.DMA for _ in range(_K)],
        ],
    )
    def k(dst_hbm, w_hbm, zeros_hbm, out_hbm, dsts_v, w_v, acc_sh, sems):
        c = lax.axis_index("c")
        s = lax.axis_index("s")
        t = c * _NSUB + s
        cp_d = pltpu.async_copy(dst_hbm.at[t], dsts_v, sems[0])
        cp_w = pltpu.async_copy(w_hbm.at[t], w_v, sems[1])

        @pl.when(s == 0)
        def _():
            pltpu.sync_copy(zeros_hbm, acc_sh)
        cp_d.wait()
        cp_w.wait()
        plsc.subcore_barrier()

        def group_body(gi, carry):
            j0 = gi * _K
            handles = [pltpu.async_copy(
                w_v.at[j0 + b], acc_sh.at[dsts_v.at[j0 + b]], sems[b],
                add=True) for b in range(_K)]
            for h in handles:
                h.wait()
            return carry
        n_groups = jnp.where(c == 0, nc1 // _K, nc0 // _K)
        lax.fori_loop(0, n_groups, group_body, 0)
        plsc.subcore_barrier()

        @pl.when(s == 0)
        def _():
            pltpu.sync_copy(acc_sh, out_hbm.at[c])

    return k


def _split_tiles(a3, nc0, nc1):
    p0 = a3[:_NSUB * nc0].reshape(_NSUB, nc0, _CHUNK)
    p1 = a3[_NSUB * nc0:].reshape(_NSUB, nc1, _CHUNK)
    p1 = jnp.pad(p1, ((0, 0), (0, nc0 - nc1), (0, 0)))
    return jnp.concatenate([p1, p0], axis=0)


def _prep_edges(src, dst, w):
    e = src.shape[0]
    tot = -(-e // _CHUNK)
    nc0 = -(-tot * 13 // (20 * _NSUB))
    nc0 = -(-nc0 // _K) * _K
    nc1 = -(-(tot - _NSUB * nc0) // _NSUB)
    nc1 = max(_K, -(-nc1 // _K) * _K)
    pad = _NSUB * (nc0 + nc1) * _CHUNK - e
    z32 = jnp.zeros((pad,), jnp.int32)
    chunks_s = jnp.concatenate([src, z32]).reshape(-1, _CHUNK)
    chunks_d = jnp.concatenate([dst, z32]).reshape(-1, _CHUNK)
    chunks_w = jnp.concatenate([w, jnp.zeros((pad,), jnp.float32)]
                               ).reshape(-1, _CHUNK)
    return (_split_tiles(chunks_s, nc0, nc1),
            _split_tiles(chunks_d, nc0, nc1),
            _split_tiles(chunks_w, nc0, nc1), nc0, nc1)


def kernel(x, edge_index, edge_attr, batch, W1, b1, g1, be1, W2, b2, g2, be2,
           W3, b3, g3, be3, W4, b4, g4, be4, W5, b5, g5, be5, Wf1, bf1,
           Wf2, bf2, Wo, bo):
    src, dst = edge_index[0], edge_index[1]
    w = edge_attr
    src3, dst3, w3, nc0, nc1 = _prep_edges(src, dst, w)
    w4 = w3.reshape(_NW, nc0, _CHUNK, 1)

    deg_parts = _sc_deg(nc0, nc1)(
        dst3, w4, jnp.zeros((N_NODES, 1), jnp.float32))
    dinv, z1p = _tc_pre(deg_parts, x, W1)

    w_flat = w3.reshape(_NW, nc0 * _CHUNK)

    def agg(zp, d):
        return _sc_edge_agg(d, nc0, nc1)(
            zp, src3, dst3, w_flat, jnp.zeros((624, d), jnp.float32))

    s1 = agg(z1p, 16)
    z2p = _tc_mid("W", s1, z1p, dinv, None, b1, g1, be1, W2, 16)
    s2 = agg(z2p, 16)
    u3p = _tc_mid("A", s2, z2p, dinv, None, b2, g2, be2, None, 16)
    s3 = agg(u3p, 16)
    u4p = _tc_mid("A", s3, u3p, dinv, W3, b3, g3, be3, None, 32)
    s4 = agg(u4p, 32)
    u5p = _tc_mid("A", s4, u4p, dinv, W4, b4, g4, be4, None, 64)
    s5 = agg(u5p, 64)
    return _tc_final(s5, u5p, dinv, W5, b5, g5, be5, batch,
                     Wf1, bf1, Wf2, bf2, Wo, bo)


# speedup vs baseline: 1.0775x; 1.0775x over previous
import functools

import jax
import jax.numpy as jnp
from jax import lax
from jax.experimental import pallas as pl
from jax.experimental.pallas import tpu as pltpu
from jax.experimental.pallas import tpu_sc as plsc

N_NODES = 10000
N_GRAPHS = 256
NEG_SLOPE = 0.01

_NCORE = 2
_NSUB = 16
_NW = _NCORE * _NSUB
_CHUNK = 128
_ROWS_PT = N_NODES // _NSUB


def _lrelu(v):
    return jnp.where(v >= 0, v, NEG_SLOPE * v)


def _bn(v, g, be):
    mu = jnp.mean(v, axis=0, keepdims=True)
    var = jnp.mean((v - mu) ** 2, axis=0, keepdims=True)
    return g[None, :] * (v - mu) * lax.rsqrt(var + 1e-5) + be[None, :]


def _tc_pre_body(deg_ref, x_ref, w1_ref, dinv_ref, z1p_ref):
    deg = 1.0 + deg_ref[0] + deg_ref[1]
    dinv = lax.rsqrt(deg)
    dinv_ref[...] = dinv
    z1 = jnp.dot(x_ref[...], w1_ref[...], preferred_element_type=jnp.float32)
    z1p_ref[...] = dinv * z1


def _tc_pre(deg, x, W1):
    return pl.pallas_call(
        _tc_pre_body,
        out_shape=(
            jax.ShapeDtypeStruct((N_NODES, 1), jnp.float32),
            jax.ShapeDtypeStruct((N_NODES, W1.shape[1]), jnp.float32),
        ),
    )(deg, x, W1)


def _tc_mid_body(mode_next, s_ref, zp_ref, dinv_ref, W_ref, b_ref, g_ref,
                 be_ref, Wn_ref, out_ref):
    dinv = dinv_ref[...]
    pre = dinv * (s_ref[0] + s_ref[1] + zp_ref[...])
    if W_ref is not None:
        pre = jnp.dot(pre, W_ref[...], preferred_element_type=jnp.float32)
    h = _lrelu(_bn(pre + b_ref[...][None, :], g_ref[...], be_ref[...]))
    if mode_next == "W":
        out_ref[...] = dinv * jnp.dot(h, Wn_ref[...],
                                      preferred_element_type=jnp.float32)
    else:
        out_ref[...] = dinv * h


def _tc_mid(mode_next, s, zp, dinv, W, b, g, be, Wn, dout):
    body = functools.partial(_tc_mid_body, mode_next)
    args = [s, zp, dinv]
    if W is None:
        def body2(s_ref, zp_ref, dinv_ref, b_ref, g_ref, be_ref, *rest):
            if mode_next == "W":
                Wn_ref, out_ref = rest
            else:
                (out_ref,) = rest
                Wn_ref = None
            _tc_mid_body(mode_next, s_ref, zp_ref, dinv_ref, None, b_ref,
                         g_ref, be_ref, Wn_ref, out_ref)
        args += [b, g, be]
        if mode_next == "W":
            args += [Wn]
        return pl.pallas_call(
            body2,
            out_shape=jax.ShapeDtypeStruct((N_NODES, dout), jnp.float32),
        )(*args)
    else:
        def body3(s_ref, zp_ref, dinv_ref, W_ref, b_ref, g_ref, be_ref, *rest):
            if mode_next == "W":
                Wn_ref, out_ref = rest
            else:
                (out_ref,) = rest
                Wn_ref = None
            _tc_mid_body(mode_next, s_ref, zp_ref, dinv_ref, W_ref, b_ref,
                         g_ref, be_ref, Wn_ref, out_ref)
        args += [W, b, g, be]
        if mode_next == "W":
            args += [Wn]
        return pl.pallas_call(
            body3,
            out_shape=jax.ShapeDtypeStruct((N_NODES, dout), jnp.float32),
        )(*args)


def _tc_final_body(s_ref, up_ref, dinv_ref, W5_ref, b5_ref, g5_ref, be5_ref,
                   batch_ref, Wf1_ref, bf1_ref, Wf2_ref, bf2_ref, Wo_ref,
                   bo_ref, out_ref):
    dinv = dinv_ref[...]
    pre = dinv * (s_ref[0] + s_ref[1] + up_ref[...])
    h = jnp.dot(pre, W5_ref[...], preferred_element_type=jnp.float32)
    h = _lrelu(_bn(h + b5_ref[...][None, :], g5_ref[...], be5_ref[...]))
    gids = lax.broadcasted_iota(jnp.int32, (N_GRAPHS, N_NODES), 0)
    mask = (gids == batch_ref[...].reshape(1, N_NODES)).astype(jnp.float32)
    pooled = jnp.dot(mask, h, preferred_element_type=jnp.float32)
    cnt = jnp.sum(mask, axis=1, keepdims=True)
    pooled = pooled / jnp.maximum(cnt, 1.0)
    f = _lrelu(jnp.dot(pooled, Wf1_ref[...],
                       preferred_element_type=jnp.float32) + bf1_ref[...][None, :])
    f = _lrelu(jnp.dot(f, Wf2_ref[...],
                       preferred_element_type=jnp.float32) + bf2_ref[...][None, :])
    out_ref[...] = jnp.dot(f, Wo_ref[...],
                           preferred_element_type=jnp.float32) + bo_ref[...][None, :]


def _tc_final(s, up, dinv, W5, b5, g5, be5, batch, Wf1, bf1, Wf2, bf2, Wo, bo):
    return pl.pallas_call(
        _tc_final_body,
        out_shape=jax.ShapeDtypeStruct((N_GRAPHS, 1), jnp.float32),
    )(s, up, dinv, W5, b5, g5, be5, batch.reshape(1, N_NODES), Wf1, bf1,
      Wf2, bf2, Wo, bo)


_K = 8


@functools.cache
def _sc_edge_agg(d, nc0, nc1):
    _K = 4 if d >= 64 else 8
    assert nc0 % _K == 0 and nc1 % _K == 0
    mesh = plsc.VectorSubcoreMesh(core_axis_name="c", subcore_axis_name="s")

    @functools.partial(
        pl.kernel, mesh=mesh,
        compiler_params=pltpu.CompilerParams(use_tc_tiling_on_sc=False),
        out_type=jax.ShapeDtypeStruct((_NCORE, N_NODES, d), jnp.float32),
        scratch_types=[
            pltpu.VMEM((nc0, _CHUNK), jnp.int32),
            pltpu.VMEM((nc0, _CHUNK), jnp.int32),
            pltpu.VMEM((nc0 * _CHUNK,), jnp.float32),
            [pltpu.VMEM((_CHUNK, d), jnp.float32) for _ in range(_K)],
            pltpu.VMEM_SHARED((N_NODES, d), jnp.float32),
            [pltpu.SemaphoreType.DMA for _ in range(_K)],
            [pltpu.SemaphoreType.DMA for _ in range(_K)],
        ],
    )
    def k(zp_hbm, src_hbm, dst_hbm, w_hbm, zeros_hbm, out_hbm,
          srcs_v, dsts_v, w_v, rowbufs, acc_sh, gsems, ssems):
        c = lax.axis_index("c")
        s = lax.axis_index("s")
        t = c * _NSUB + s
        cp_s = pltpu.async_copy(src_hbm.at[t], srcs_v, gsems[0])
        cp_d = pltpu.async_copy(dst_hbm.at[t], dsts_v, gsems[1])
        cp_w = pltpu.async_copy(w_hbm.at[t], w_v, gsems[2])
        pltpu.sync_copy(zeros_hbm, acc_sh.at[pl.ds(s * 624, 624)])

        @pl.when(s == 0)
        def _():
            pltpu.sync_copy(zeros_hbm.at[pl.ds(0, 16)],
                            acc_sh.at[pl.ds(_NSUB * 624, 16)])
        cp_s.wait()
        cp_d.wait()
        cp_w.wait()
        plsc.subcore_barrier()

        def group_body(gi, carry):
            j0 = gi * _K
            gathers = []
            for b in range(_K):
                gathers.append(pltpu.async_copy(
                    zp_hbm.at[srcs_v.at[j0 + b]], rowbufs[b], gsems[b]))
            scatters = []
            for b in range(_K):
                gathers[b].wait()
                rows_v = rowbufs[b]
                for g in range(_CHUNK // 16):
                    wv16 = w_v[pl.ds((j0 + b) * _CHUNK + g * 16, 16)]
                    for i in range(16):
                        e = g * 16 + i
                        wv = wv16[i]
                        for kk in range(d // 16):
                            rows_v[e, pl.ds(kk * 16, 16)] = (
                                rows_v[e, pl.ds(kk * 16, 16)] * wv)
                scatters.append(pltpu.async_copy(
                    rows_v, acc_sh.at[dsts_v.at[j0 + b]], ssems[b], add=True))
            for b in range(_K):
                scatters[b].wait()
            return carry
        lax.fori_loop(0, nc0 // _K, group_body, 0)
        plsc.subcore_barrier()
        pltpu.sync_copy(acc_sh.at[pl.ds(s * 624, 624)],
                        out_hbm.at[c, pl.ds(s * 624, 624)])

        @pl.when(s == 0)
        def _():
            pltpu.sync_copy(acc_sh.at[pl.ds(_NSUB * 624, 16)],
                            out_hbm.at[c, pl.ds(_NSUB * 624, 16)])

    return k


@functools.cache
def _sc_deg(nc0, nc1):
    mesh = plsc.VectorSubcoreMesh(core_axis_name="c", subcore_axis_name="s")

    @functools.partial(
        pl.kernel, mesh=mesh,
        compiler_params=pltpu.CompilerParams(use_tc_tiling_on_sc=False),
        out_type=jax.ShapeDtypeStruct((_NCORE, N_NODES, 1), jnp.float32),
        scratch_types=[
            pltpu.VMEM((nc0, _CHUNK), jnp.int32),
            pltpu.VMEM((nc0, _CHUNK, 1), jnp.float32),
            pltpu.VMEM_SHARED((N_NODES, 1), jnp.float32),
            [pltpu.SemaphoreType.DMA for _ in range(_K)],
        ],
    )
    def k(dst_hbm, w_hbm, zeros_hbm, out_hbm, dsts_v, w_v, acc_sh, sems):
        c = lax.axis_index("c")
        s = lax.axis_index("s")
        t = c * _NSUB + s
        cp_d = pltpu.async_copy(dst_hbm.at[t], dsts_v, sems[0])
        cp_w = pltpu.async_copy(w_hbm.at[t], w_v, sems[1])

        @pl.when(s == 0)
        def _():
            pltpu.sync_copy(zeros_hbm, acc_sh)
        cp_d.wait()
        cp_w.wait()
        plsc.subcore_barrier()

        def group_body(gi, carry):
            j0 = gi * _K
            handles = [pltpu.async_copy(
                w_v.at[j0 + b], acc_sh.at[dsts_v.at[j0 + b]], sems[b],
                add=True) for b in range(_K)]
            for h in handles:
                h.wait()
            return carry
        lax.fori_loop(0, nc0 // _K, group_body, 0)
        plsc.subcore_barrier()

        @pl.when(s == 0)
        def _():
            pltpu.sync_copy(acc_sh, out_hbm.at[c])

    return k


def _prep_edges(src, dst, w):
    e = src.shape[0]
    n_chunks = -(-e // (_NW * _CHUNK))
    n_chunks = -(-n_chunks // _K) * _K
    pad = _NW * n_chunks * _CHUNK - e
    z32 = jnp.zeros((pad,), jnp.int32)
    src3 = jnp.concatenate([src, z32]).reshape(_NW, n_chunks, _CHUNK)
    dst3 = jnp.concatenate([dst, z32]).reshape(_NW, n_chunks, _CHUNK)
    w3 = jnp.concatenate([w, jnp.zeros((pad,), jnp.float32)]
                         ).reshape(_NW, n_chunks, _CHUNK)
    return src3, dst3, w3, n_chunks, n_chunks


def kernel(x, edge_index, edge_attr, batch, W1, b1, g1, be1, W2, b2, g2, be2,
           W3, b3, g3, be3, W4, b4, g4, be4, W5, b5, g5, be5, Wf1, bf1,
           Wf2, bf2, Wo, bo):
    src, dst = edge_index[0], edge_index[1]
    w = edge_attr
    src3, dst3, w3, nc0, nc1 = _prep_edges(src, dst, w)
    w4 = w3.reshape(_NW, nc0, _CHUNK, 1)

    deg_parts = _sc_deg(nc0, nc1)(
        dst3, w4, jnp.zeros((N_NODES, 1), jnp.float32))
    dinv, z1p = _tc_pre(deg_parts, x, W1)

    w_flat = w3.reshape(_NW, nc0 * _CHUNK)

    def agg(zp, d):
        return _sc_edge_agg(d, nc0, nc1)(
            zp, src3, dst3, w_flat, jnp.zeros((624, d), jnp.float32))

    s1 = agg(z1p, 16)
    z2p = _tc_mid("W", s1, z1p, dinv, None, b1, g1, be1, W2, 16)
    s2 = agg(z2p, 16)
    u3p = _tc_mid("A", s2, z2p, dinv, None, b2, g2, be2, None, 16)
    s3 = agg(u3p, 16)
    u4p = _tc_mid("A", s3, u3p, dinv, W3, b3, g3, be3, None, 32)
    s4 = agg(u4p, 32)
    u5p = _tc_mid("A", s4, u4p, dinv, W4, b4, g4, be4, None, 64)
    s5 = agg(u5p, 64)
    return _tc_final(s5, u5p, dinv, W5, b5, g5, be5, batch,
                     Wf1, bf1, Wf2, bf2, Wo, bo)
